# Initial kernel scaffold; baseline (speedup 1.0000x reference)
import jax
import jax.numpy as jnp
from jax.experimental import pallas as pl


def kernel(x, emb, Wp, bp, Wf, bf):
    raise NotImplementedError("write your pallas kernel here")


# speedup vs baseline: 2.2702x; 2.2702x over previous
import functools

import jax
import jax.numpy as jnp
from jax import lax
from jax.experimental import pallas as pl
from jax.experimental.pallas import tpu as pltpu
from jax.experimental.pallas import tpu_sc as plsc

B, L, D = 4096, 200, 128
NL = 10
LP = 208
CH = LP // 2
NC, NS, LANES = 2, 16, 16
NW = NC * NS
RPW = B // NW
NVR = D // LANES


def _make_bag():
    mesh = plsc.VectorSubcoreMesh(core_axis_name="c", subcore_axis_name="s")

    @functools.partial(
        pl.kernel,
        mesh=mesh,
        out_type=jax.ShapeDtypeStruct((B, D), jnp.float32),
        scratch_types=[
            pltpu.VMEM((RPW * LP,), jnp.int32),
            pltpu.VMEM((LP, D), jnp.float32),
            pltpu.VMEM((RPW, D), jnp.float32),
            pltpu.SemaphoreType.DMA,
        ],
    )
    def bag(x_hbm, emb_hbm, out_hbm, idx_v, rows_v, z_v, sem):
        wid = lax.axis_index("s") * NC + lax.axis_index("c")
        base = wid * RPW
        pltpu.sync_copy(
            x_hbm.at[pl.ds(pl.multiple_of(base * LP, 8), RPW * LP)], idx_v)

        def row_body(b, carry):
            off = pl.multiple_of(b * LP, 8)
            c0 = pltpu.async_copy(
                emb_hbm.at[idx_v.at[pl.ds(off, CH)]], rows_v.at[pl.ds(0, CH)], sem)
            c1 = pltpu.async_copy(
                emb_hbm.at[idx_v.at[pl.ds(off + CH, CH)]], rows_v.at[pl.ds(CH, CH)], sem)
            c0.wait()
            c1.wait()

            def red(r, accs):
                return tuple(
                    a + rows_v[r, pl.ds(j * LANES, LANES)]
                    for j, a in enumerate(accs))

            accs = lax.fori_loop(
                0, LP, red,
                tuple(jnp.zeros((LANES,), jnp.float32) for _ in range(NVR)))
            for j in range(NVR):
                z_v[b, pl.ds(j * LANES, LANES)] = accs[j]
            return carry

        lax.fori_loop(0, RPW, row_body, 0)
        pltpu.sync_copy(z_v, out_hbm.at[pl.ds(base, RPW)])

    return bag


_bag = _make_bag()


def _head_body(s_ref, x_ref, wp_ref, bp_ref, wf_ref, bf_ref, o_ref):
    cnt = jnp.sum((x_ref[...] != 0).astype(jnp.float32), axis=1, keepdims=True)
    z = s_ref[...] / jnp.maximum(cnt, 1.0)
    h = lax.dot_general(z, wp_ref[...], (((1,), (1,)), ((), ())),
                        preferred_element_type=jnp.float32)
    h = jnp.maximum(h + bp_ref[...], 0.0)
    o = lax.dot_general(h, wf_ref[...], (((1,), (1,)), ((), ())),
                        preferred_element_type=jnp.float32)
    o_ref[...] = o + bf_ref[...]


BT = 512


_head = pl.pallas_call(
    _head_body,
    grid=(B // BT,),
    in_specs=[
        pl.BlockSpec((BT, D), lambda i: (i, 0)),
        pl.BlockSpec((BT, L), lambda i: (i, 0)),
        pl.BlockSpec((D, D), lambda i: (0, 0)),
        pl.BlockSpec((1, D), lambda i: (0, 0)),
        pl.BlockSpec((NL, D), lambda i: (0, 0)),
        pl.BlockSpec((1, NL), lambda i: (0, 0)),
    ],
    out_specs=pl.BlockSpec((BT, NL), lambda i: (i, 0)),
    out_shape=jax.ShapeDtypeStruct((B, NL), jnp.float32),
)


def kernel(x, emb, Wp, bp, Wf, bf):
    x_pad = jnp.pad(x, ((0, 0), (0, LP - L))).reshape(B * LP)
    sums = _bag(x_pad, emb)
    return _head(sums, x, Wp, bp.reshape(1, D), Wf, bf.reshape(1, NL))
